# baseline (speedup 1.0000x reference)
import functools

import jax
import jax.numpy as jnp
from jax.experimental import pallas as pl
from jax.experimental.pallas import tpu as pltpu

_LN_EPS = 1e-5
_NCORES = 2


def _fused_kernel(p_hbm, f_hbm, a_hbm, w_in_ref, b_in_ref,
                  g_in_ref, be_in_ref, g_out_ref, be_out_ref,
                  w_head_ref, b_head_ref, out_ref,
                  pbuf, fbuf, abuf, z_ref, psem, fsem, asem,
                  *, NC, G, H, N, d_in, d_model):
    c = pl.program_id(0)
    GH = G * H
    GHN = G * H * N

    def start(k):
        idx = c * NC + k
        slot = k % 2
        pltpu.make_async_copy(p_hbm.at[pl.ds(idx * GH, GH)],
                              pbuf.at[slot], psem.at[slot]).start()
        pltpu.make_async_copy(f_hbm.at[pl.ds(idx * GHN, GHN)],
                              fbuf.at[slot], fsem.at[slot]).start()
        pltpu.make_async_copy(a_hbm.at[pl.ds(idx * GHN, GHN)],
                              abuf.at[slot], asem.at[slot]).start()

    def wait(k):
        idx = c * NC + k
        slot = k % 2
        pltpu.make_async_copy(p_hbm.at[pl.ds(idx * GH, GH)],
                              pbuf.at[slot], psem.at[slot]).wait()
        pltpu.make_async_copy(f_hbm.at[pl.ds(idx * GHN, GHN)],
                              fbuf.at[slot], fsem.at[slot]).wait()
        pltpu.make_async_copy(a_hbm.at[pl.ds(idx * GHN, GHN)],
                              abuf.at[slot], asem.at[slot]).wait()

    start(0)
    if NC > 1:
        start(1)

    for k in range(NC):
        wait(k)
        if 1 <= k < NC - 1:
            start(k + 1)
        slot = k % 2

        f = fbuf[slot]
        a = abuf[slot]
        y = (jnp.dot(f, w_in_ref[0:d_in, :],
                     preferred_element_type=jnp.float32)
             + jnp.dot(a, w_in_ref[d_in:2 * d_in, :],
                       preferred_element_type=jnp.float32)
             + b_in_ref[...])

        mu = jnp.mean(y, axis=-1, keepdims=True)
        var = jnp.mean(y * y, axis=-1, keepdims=True) - mu * mu
        y = ((y - mu) * jax.lax.rsqrt(var + _LN_EPS) * g_in_ref[...]
             + be_in_ref[...])
        y = jnp.maximum(y, 0.0)

        stats = []
        for g in range(G):
            s = None
            q = None
            for h in range(H):
                i = g * H + h
                p = pbuf[slot, i]
                sf = jnp.dot(p, y[i * N:(i + 1) * N, :],
                             preferred_element_type=jnp.float32)
                ob = jax.lax.dot_general(p, sf, (((0,), (0,)), ((), ())),
                                         preferred_element_type=jnp.float32)
                z_ref[g * N:(g + 1) * N,
                      h * d_model:(h + 1) * d_model] = ob
                rs = jnp.sum(ob, axis=-1, keepdims=True)
                rq = jnp.sum(ob * ob, axis=-1, keepdims=True)
                s = rs if s is None else s + rs
                q = rq if q is None else q + rq
            stats.append((s, q))

        inv_hd = 1.0 / (H * d_model)
        mu2 = jnp.concatenate([s for s, _ in stats], axis=0) * inv_hd
        q2 = jnp.concatenate([q for _, q in stats], axis=0) * inv_hd
        rstd = jax.lax.rsqrt(q2 - mu2 * mu2 + _LN_EPS)

        z = z_ref[...]
        zn = (z - mu2) * rstd * g_out_ref[...] + be_out_ref[...]
        out_ref[k * G * N:(k + 1) * G * N, :] = (
            jnp.dot(zn, w_head_ref[...], preferred_element_type=jnp.float32)
            + b_head_ref[...])


def kernel(perm, adj, features, appd, w_in, b_in, ln_in_g, ln_in_b,
           ln_out_g, ln_out_b, w_head, b_head):
    del adj
    B, H, N, _ = perm.shape
    d_in = features.shape[-1]
    d_model = w_in.shape[1]
    nclass = w_head.shape[1]

    G = min(4, B // _NCORES)
    NC = B // (_NCORES * G)

    p2 = perm.reshape(B * H, N, N)
    f2 = features.reshape(B * H * N, d_in)
    a2 = appd.reshape(B * H * N, d_in)

    fused = functools.partial(_fused_kernel, NC=NC, G=G, H=H, N=N,
                              d_in=d_in, d_model=d_model)
    out = pl.pallas_call(
        fused,
        out_shape=jax.ShapeDtypeStruct((B * N, nclass), jnp.float32),
        grid=(_NCORES,),
        in_specs=[
            pl.BlockSpec(memory_space=pl.ANY),
            pl.BlockSpec(memory_space=pl.ANY),
            pl.BlockSpec(memory_space=pl.ANY),
            pl.BlockSpec((2 * d_in, d_model), lambda c: (0, 0)),
            pl.BlockSpec((1, d_model), lambda c: (0, 0)),
            pl.BlockSpec((1, d_model), lambda c: (0, 0)),
            pl.BlockSpec((1, d_model), lambda c: (0, 0)),
            pl.BlockSpec((1, H * d_model), lambda c: (0, 0)),
            pl.BlockSpec((1, H * d_model), lambda c: (0, 0)),
            pl.BlockSpec((H * d_model, nclass), lambda c: (0, 0)),
            pl.BlockSpec((1, nclass), lambda c: (0, 0)),
        ],
        out_specs=pl.BlockSpec((B * N // _NCORES, nclass),
                               lambda c: (c, 0)),
        scratch_shapes=[
            pltpu.VMEM((2, G * H, N, N), jnp.float32),
            pltpu.VMEM((2, G * H * N, d_in), jnp.float32),
            pltpu.VMEM((2, G * H * N, d_in), jnp.float32),
            pltpu.VMEM((G * N, H * d_model), jnp.float32),
            pltpu.SemaphoreType.DMA((2,)),
            pltpu.SemaphoreType.DMA((2,)),
            pltpu.SemaphoreType.DMA((2,)),
        ],
        compiler_params=pltpu.CompilerParams(
            dimension_semantics=("parallel",)),
    )(p2, f2, a2, w_in, b_in, ln_in_g, ln_in_b,
      ln_out_g, ln_out_b, w_head, b_head)
    return out.reshape(B, N, nclass)
